# baseline (speedup 1.0000x reference)
import jax
import jax.numpy as jnp
from jax.experimental import pallas as pl
from jax.experimental.pallas import tpu as pltpu


def _pick_group(b):
    for g in (32, 16, 8, 4, 2):
        if b % g == 0:
            return g
    return 1


def _make_gin_kernel(L, G, N, D, Din, H, packed_view):
    o_w1 = Din
    o_w2 = o_w1 + L * D
    o_bf = o_w2 + L * H
    o_b1 = o_bf + 1
    o_b2 = o_b1 + L

    def body(adj_ref, feat_ref, wts_ref, eps_ref,
             e0_ref, e1_ref, e2_ref, e3_ref, stacked_ref,
             p0_ref, p1_ref, p2_ref, p3_ref, pv_ref):
        e_refs = (e0_ref, e1_ref, e2_ref, e3_ref)
        p_refs = (p0_ref, p1_ref, p2_ref, p3_ref)
        wts = wts_ref[...]

        def mm(a, b):
            return jnp.dot(a, b, preferred_element_type=jnp.float32)

        def emit(l, x):
            x3 = x.reshape(G, N, D)
            e_refs[l][...] = x3
            stacked_ref[l] = x3
            pool = jnp.sum(x3, axis=1)
            p_refs[l][...] = pool
            if packed_view:
                pv_ref[l] = pool.reshape(G // 4, 4 * D)
            else:
                pv_ref[l] = pool

        feat2d = feat_ref[...].reshape(G * N, -1)
        x = mm(feat2d, wts[0:Din]) + wts[o_bf:o_bf + 1]
        emit(0, x)

        for l in range(L):
            s = 1.0 + eps_ref[l]
            aggs = [mm(adj_ref[g], x[g * N:(g + 1) * N]) for g in range(G)]
            agg = jnp.concatenate(aggs, axis=0) + s * x
            h = jnp.maximum(
                mm(agg, wts[o_w1 + l * D:o_w1 + (l + 1) * D])
                + wts[o_b1 + l:o_b1 + l + 1], 0.0)
            x = jnp.maximum(
                mm(h, wts[o_w2 + l * H:o_w2 + (l + 1) * H])
                + wts[o_b2 + l:o_b2 + l + 1], 0.0)
            emit(l + 1, x)

    return body


def _head_kernel(pv_ref, wl1_ref, bl1_ref, wl2_ref, bl2_ref,
                 l1_ref, l2_ref, sm_ref):
    h = jnp.dot(pv_ref[...], wl1_ref[...],
                preferred_element_type=jnp.float32) + bl1_ref[...]
    h = jnp.maximum(h, 0.0)
    l1_ref[...] = h
    z = jnp.dot(h, wl2_ref[...],
                preferred_element_type=jnp.float32) + bl2_ref[...]
    l2_ref[...] = z
    e = jnp.exp(z - jnp.max(z, axis=-1, keepdims=True))
    sm_ref[...] = e / jnp.sum(e, axis=-1, keepdims=True)


def kernel(adj, feat, w_first, b_first, eps, w1_all, b1_all, w2_all, b2_all,
           w_lin1, b_lin1, w_lin2, b_lin2):
    B, N, Din = feat.shape
    D = w_first.shape[1]
    L, _, H = w1_all.shape
    C = w_lin2.shape[1]
    F_cat = (L + 1) * D
    F_mid = w_lin1.shape[1]
    G = _pick_group(B)
    packed_view = (G % 32 == 0) and (F_cat == 4 * D) and (B % 4 == 0)

    smem = pl.BlockSpec(memory_space=pltpu.MemorySpace.SMEM)

    n_rows = Din + L * D + L * H + 1 + 2 * L
    pad = (-n_rows) % 8
    wts_packed = jnp.concatenate(
        [w_first, w1_all.reshape(L * D, H), w2_all.reshape(L * H, D),
         b_first, b1_all.reshape(L, H), b2_all.reshape(L, D)]
        + ([jnp.zeros((pad, D), jnp.float32)] if pad else []), axis=0)

    if packed_view:
        pv_spec = pl.BlockSpec((L + 1, G // 4, 4 * D), lambda b: (0, b, 0))
        pv_shape = jax.ShapeDtypeStruct((L + 1, B // 4, 4 * D), jnp.float32)
    else:
        pv_spec = pl.BlockSpec((L + 1, G, D), lambda b: (0, b, 0))
        pv_shape = jax.ShapeDtypeStruct((L + 1, B, D), jnp.float32)

    outs = pl.pallas_call(
        _make_gin_kernel(L, G, N, D, Din, H, packed_view),
        grid=(B // G,),
        in_specs=[
            pl.BlockSpec((G, N, N), lambda b: (b, 0, 0)),
            pl.BlockSpec((G, N, Din), lambda b: (b, 0, 0)),
            pl.BlockSpec((n_rows + pad, D), lambda b: (0, 0)),
            smem,
        ],
        out_specs=(
            pl.BlockSpec((G, N, D), lambda b: (b, 0, 0)),
            pl.BlockSpec((G, N, D), lambda b: (b, 0, 0)),
            pl.BlockSpec((G, N, D), lambda b: (b, 0, 0)),
            pl.BlockSpec((G, N, D), lambda b: (b, 0, 0)),
            pl.BlockSpec((L + 1, G, N, D), lambda b: (0, b, 0, 0)),
            pl.BlockSpec((G, D), lambda b: (b, 0)),
            pl.BlockSpec((G, D), lambda b: (b, 0)),
            pl.BlockSpec((G, D), lambda b: (b, 0)),
            pl.BlockSpec((G, D), lambda b: (b, 0)),
            pv_spec,
        ),
        out_shape=(
            jax.ShapeDtypeStruct((B, N, D), jnp.float32),
            jax.ShapeDtypeStruct((B, N, D), jnp.float32),
            jax.ShapeDtypeStruct((B, N, D), jnp.float32),
            jax.ShapeDtypeStruct((B, N, D), jnp.float32),
            jax.ShapeDtypeStruct((L + 1, B, N, D), jnp.float32),
            jax.ShapeDtypeStruct((B, D), jnp.float32),
            jax.ShapeDtypeStruct((B, D), jnp.float32),
            jax.ShapeDtypeStruct((B, D), jnp.float32),
            jax.ShapeDtypeStruct((B, D), jnp.float32),
            pv_shape,
        ),
        compiler_params=pltpu.CompilerParams(
            dimension_semantics=("parallel",)),
    )(adj, feat, wts_packed, eps)

    e0, e1, e2, e3, embeds_stacked, p0, p1, p2, p3, pv = outs

    pooled_view = pv.reshape(B, F_cat)
    mlps_output_embeds_pooled = [p0, p1, p2, p3][:L + 1]

    vmem = pl.BlockSpec(memory_space=pltpu.MemorySpace.VMEM)
    lin1_out, lin2_out, softmax_out = pl.pallas_call(
        _head_kernel,
        in_specs=[vmem] * 5,
        out_specs=(vmem, vmem, vmem),
        out_shape=(
            jax.ShapeDtypeStruct((B, F_mid), jnp.float32),
            jax.ShapeDtypeStruct((B, C), jnp.float32),
            jax.ShapeDtypeStruct((B, C), jnp.float32),
        ),
    )(pooled_view, w_lin1, b_lin1, w_lin2, b_lin2)

    mlps_output_embeds = [e0, e1, e2, e3][:L + 1]

    return (mlps_output_embeds, embeds_stacked, mlps_output_embeds_pooled,
            pooled_view, lin1_out, lin1_out, lin2_out, softmax_out)


# speedup vs baseline: 1.1055x; 1.1055x over previous
import jax
import jax.numpy as jnp
from jax.experimental import pallas as pl
from jax.experimental.pallas import tpu as pltpu


def _pick_group(b):
    for g in (32, 16, 8, 4, 2):
        if b % g == 0:
            return g
    return 1


def _make_gin_kernel(L, G, N, D, packed_view):
    def body(adj_ref, feat_ref, wf_ref, bf_ref, eps_ref,
             w1_ref, b1_ref, w2_ref, b2_ref,
             e0_ref, e1_ref, e2_ref, e3_ref, stacked_ref,
             p0_ref, p1_ref, p2_ref, p3_ref, pv_ref):
        e_refs = (e0_ref, e1_ref, e2_ref, e3_ref)
        p_refs = (p0_ref, p1_ref, p2_ref, p3_ref)

        def mm(a, b):
            return jnp.dot(a, b, preferred_element_type=jnp.float32)

        def emit(l, x):
            x3 = x.reshape(G, N, D)
            e_refs[l][...] = x3
            stacked_ref[l] = x3
            pool = jnp.sum(x3, axis=1)
            p_refs[l][...] = pool
            if packed_view:
                pv_ref[l] = pool.reshape(G // 4, 4 * D)
            else:
                pv_ref[l] = pool

        feat2d = feat_ref[...].reshape(G * N, -1)
        x = mm(feat2d, wf_ref[...]) + bf_ref[...]
        emit(0, x)

        for l in range(L):
            s = 1.0 + eps_ref[l]
            aggs = [mm(adj_ref[g], x[g * N:(g + 1) * N]) for g in range(G)]
            agg = jnp.concatenate(aggs, axis=0) + s * x
            h = jnp.maximum(mm(agg, w1_ref[l]) + b1_ref[l], 0.0)
            x = jnp.maximum(mm(h, w2_ref[l]) + b2_ref[l], 0.0)
            emit(l + 1, x)

    return body


def _head_kernel(pv_ref, wl1_ref, bl1_ref, wl2_ref, bl2_ref,
                 l1_ref, l2_ref, sm_ref):
    h = jnp.dot(pv_ref[...], wl1_ref[...],
                preferred_element_type=jnp.float32) + bl1_ref[...]
    h = jnp.maximum(h, 0.0)
    l1_ref[...] = h
    z = jnp.dot(h, wl2_ref[...],
                preferred_element_type=jnp.float32) + bl2_ref[...]
    l2_ref[...] = z
    e = jnp.exp(z - jnp.max(z, axis=-1, keepdims=True))
    sm_ref[...] = e / jnp.sum(e, axis=-1, keepdims=True)


def kernel(adj, feat, w_first, b_first, eps, w1_all, b1_all, w2_all, b2_all,
           w_lin1, b_lin1, w_lin2, b_lin2):
    B, N, Din = feat.shape
    D = w_first.shape[1]
    L, _, H = w1_all.shape
    C = w_lin2.shape[1]
    F_cat = (L + 1) * D
    F_mid = w_lin1.shape[1]
    G = _pick_group(B)
    packed_view = (G % 32 == 0) and (F_cat == 4 * D) and (B % 4 == 0)

    smem = pl.BlockSpec(memory_space=pltpu.MemorySpace.SMEM)

    if packed_view:
        pv_spec = pl.BlockSpec((L + 1, G // 4, 4 * D), lambda b: (0, b, 0))
        pv_shape = jax.ShapeDtypeStruct((L + 1, B // 4, 4 * D), jnp.float32)
    else:
        pv_spec = pl.BlockSpec((L + 1, G, D), lambda b: (0, b, 0))
        pv_shape = jax.ShapeDtypeStruct((L + 1, B, D), jnp.float32)

    outs = pl.pallas_call(
        _make_gin_kernel(L, G, N, D, packed_view),
        grid=(B // G,),
        in_specs=[
            pl.BlockSpec((G, N, N), lambda b: (b, 0, 0)),
            pl.BlockSpec((G, N, Din), lambda b: (b, 0, 0)),
            pl.BlockSpec((Din, D), lambda b: (0, 0)),
            pl.BlockSpec((1, D), lambda b: (0, 0)),
            smem,
            pl.BlockSpec((L, D, H), lambda b: (0, 0, 0)),
            pl.BlockSpec((L, 1, H), lambda b: (0, 0, 0)),
            pl.BlockSpec((L, H, D), lambda b: (0, 0, 0)),
            pl.BlockSpec((L, 1, D), lambda b: (0, 0, 0)),
        ],
        out_specs=(
            pl.BlockSpec((G, N, D), lambda b: (b, 0, 0)),
            pl.BlockSpec((G, N, D), lambda b: (b, 0, 0)),
            pl.BlockSpec((G, N, D), lambda b: (b, 0, 0)),
            pl.BlockSpec((G, N, D), lambda b: (b, 0, 0)),
            pl.BlockSpec((L + 1, G, N, D), lambda b: (0, b, 0, 0)),
            pl.BlockSpec((G, D), lambda b: (b, 0)),
            pl.BlockSpec((G, D), lambda b: (b, 0)),
            pl.BlockSpec((G, D), lambda b: (b, 0)),
            pl.BlockSpec((G, D), lambda b: (b, 0)),
            pv_spec,
        ),
        out_shape=(
            jax.ShapeDtypeStruct((B, N, D), jnp.float32),
            jax.ShapeDtypeStruct((B, N, D), jnp.float32),
            jax.ShapeDtypeStruct((B, N, D), jnp.float32),
            jax.ShapeDtypeStruct((B, N, D), jnp.float32),
            jax.ShapeDtypeStruct((L + 1, B, N, D), jnp.float32),
            jax.ShapeDtypeStruct((B, D), jnp.float32),
            jax.ShapeDtypeStruct((B, D), jnp.float32),
            jax.ShapeDtypeStruct((B, D), jnp.float32),
            jax.ShapeDtypeStruct((B, D), jnp.float32),
            pv_shape,
        ),
        compiler_params=pltpu.CompilerParams(
            dimension_semantics=("parallel",)),
    )(adj, feat, w_first, b_first, eps, w1_all, b1_all, w2_all, b2_all)

    e0, e1, e2, e3, embeds_stacked, p0, p1, p2, p3, pv = outs

    pooled_view = pv.reshape(B, F_cat)
    mlps_output_embeds_pooled = [p0, p1, p2, p3][:L + 1]

    vmem = pl.BlockSpec(memory_space=pltpu.MemorySpace.VMEM)
    lin1_out, lin2_out, softmax_out = pl.pallas_call(
        _head_kernel,
        in_specs=[vmem] * 5,
        out_specs=(vmem, vmem, vmem),
        out_shape=(
            jax.ShapeDtypeStruct((B, F_mid), jnp.float32),
            jax.ShapeDtypeStruct((B, C), jnp.float32),
            jax.ShapeDtypeStruct((B, C), jnp.float32),
        ),
    )(pooled_view, w_lin1, b_lin1, w_lin2, b_lin2)

    mlps_output_embeds = [e0, e1, e2, e3][:L + 1]

    return (mlps_output_embeds, embeds_stacked, mlps_output_embeds_pooled,
            pooled_view, lin1_out, lin1_out, lin2_out, softmax_out)


# speedup vs baseline: 1.1151x; 1.0087x over previous
import jax
import jax.numpy as jnp
from jax.experimental import pallas as pl
from jax.experimental.pallas import tpu as pltpu


def _pick_group(b):
    for g in (32, 16, 8, 4, 2):
        if b % g == 0:
            return g
    return 1


def _make_gin_kernel(L, G, N, D, packed_view):
    def body(adj_ref, feat_ref, wf_ref, bf_ref, eps_ref,
             w1_ref, b1_ref, w2_ref, b2_ref,
             e0_ref, e1_ref, e2_ref, e3_ref, stacked_ref, pv_ref):
        e_refs = (e0_ref, e1_ref, e2_ref, e3_ref)

        def mm(a, b):
            return jnp.dot(a, b, preferred_element_type=jnp.float32)

        def emit(l, x):
            x3 = x.reshape(G, N, D)
            e_refs[l][...] = x3
            stacked_ref[l] = x3
            pool = jnp.sum(x3, axis=1)
            if packed_view:
                pv_ref[l] = pool.reshape(G // 4, 4 * D)
            else:
                pv_ref[l] = pool

        feat2d = feat_ref[...].reshape(G * N, -1)
        x = mm(feat2d, wf_ref[...]) + bf_ref[...]
        emit(0, x)

        for l in range(L):
            s = 1.0 + eps_ref[l]
            aggs = [mm(adj_ref[g], x[g * N:(g + 1) * N]) for g in range(G)]
            agg = jnp.concatenate(aggs, axis=0) + s * x
            h = jnp.maximum(mm(agg, w1_ref[l]) + b1_ref[l], 0.0)
            x = jnp.maximum(mm(h, w2_ref[l]) + b2_ref[l], 0.0)
            emit(l + 1, x)

    return body


def _make_head_kernel(L, B, D):
    def body(pv_ref, wl1_ref, bl1_ref, wl2_ref, bl2_ref,
             l1_ref, l2_ref, sm_ref, p0_ref, p1_ref, p2_ref, p3_ref):
        p_refs = (p0_ref, p1_ref, p2_ref, p3_ref)
        pview = pv_ref[...]
        h = jnp.dot(pview, wl1_ref[...],
                    preferred_element_type=jnp.float32) + bl1_ref[...]
        h = jnp.maximum(h, 0.0)
        l1_ref[...] = h
        z = jnp.dot(h, wl2_ref[...],
                    preferred_element_type=jnp.float32) + bl2_ref[...]
        l2_ref[...] = z
        e = jnp.exp(z - jnp.max(z, axis=-1, keepdims=True))
        sm_ref[...] = e / jnp.sum(e, axis=-1, keepdims=True)
        pv3 = pview.reshape(L + 1, B // 4, 4 * D)
        for l in range(L + 1):
            p_refs[l][...] = pv3[l].reshape(B, D)

    return body


def kernel(adj, feat, w_first, b_first, eps, w1_all, b1_all, w2_all, b2_all,
           w_lin1, b_lin1, w_lin2, b_lin2):
    B, N, Din = feat.shape
    D = w_first.shape[1]
    L, _, H = w1_all.shape
    C = w_lin2.shape[1]
    F_cat = (L + 1) * D
    F_mid = w_lin1.shape[1]
    G = _pick_group(B)
    packed_view = (G % 32 == 0) and (F_cat == 4 * D) and (B % 4 == 0)

    smem = pl.BlockSpec(memory_space=pltpu.MemorySpace.SMEM)

    if packed_view:
        pv_spec = pl.BlockSpec((L + 1, G // 4, 4 * D), lambda b: (0, b, 0))
        pv_shape = jax.ShapeDtypeStruct((L + 1, B // 4, 4 * D), jnp.float32)
    else:
        pv_spec = pl.BlockSpec((L + 1, G, D), lambda b: (0, b, 0))
        pv_shape = jax.ShapeDtypeStruct((L + 1, B, D), jnp.float32)

    outs = pl.pallas_call(
        _make_gin_kernel(L, G, N, D, packed_view),
        grid=(B // G,),
        in_specs=[
            pl.BlockSpec((G, N, N), lambda b: (b, 0, 0)),
            pl.BlockSpec((G, N, Din), lambda b: (b, 0, 0)),
            pl.BlockSpec((Din, D), lambda b: (0, 0)),
            pl.BlockSpec((1, D), lambda b: (0, 0)),
            smem,
            pl.BlockSpec((L, D, H), lambda b: (0, 0, 0)),
            pl.BlockSpec((L, 1, H), lambda b: (0, 0, 0)),
            pl.BlockSpec((L, H, D), lambda b: (0, 0, 0)),
            pl.BlockSpec((L, 1, D), lambda b: (0, 0, 0)),
        ],
        out_specs=(
            pl.BlockSpec((G, N, D), lambda b: (b, 0, 0)),
            pl.BlockSpec((G, N, D), lambda b: (b, 0, 0)),
            pl.BlockSpec((G, N, D), lambda b: (b, 0, 0)),
            pl.BlockSpec((G, N, D), lambda b: (b, 0, 0)),
            pl.BlockSpec((L + 1, G, N, D), lambda b: (0, b, 0, 0)),
            pv_spec,
        ),
        out_shape=(
            jax.ShapeDtypeStruct((B, N, D), jnp.float32),
            jax.ShapeDtypeStruct((B, N, D), jnp.float32),
            jax.ShapeDtypeStruct((B, N, D), jnp.float32),
            jax.ShapeDtypeStruct((B, N, D), jnp.float32),
            jax.ShapeDtypeStruct((L + 1, B, N, D), jnp.float32),
            pv_shape,
        ),
        compiler_params=pltpu.CompilerParams(
            dimension_semantics=("parallel",)),
    )(adj, feat, w_first, b_first, eps, w1_all, b1_all, w2_all, b2_all)

    e0, e1, e2, e3, embeds_stacked, pv = outs

    pooled_view = pv.reshape(B, F_cat)

    vmem = pl.BlockSpec(memory_space=pltpu.MemorySpace.VMEM)
    head_outs = pl.pallas_call(
        _make_head_kernel(L, B, D),
        in_specs=[vmem] * 5,
        out_specs=(vmem,) * 7,
        out_shape=(
            jax.ShapeDtypeStruct((B, F_mid), jnp.float32),
            jax.ShapeDtypeStruct((B, C), jnp.float32),
            jax.ShapeDtypeStruct((B, C), jnp.float32),
            jax.ShapeDtypeStruct((B, D), jnp.float32),
            jax.ShapeDtypeStruct((B, D), jnp.float32),
            jax.ShapeDtypeStruct((B, D), jnp.float32),
            jax.ShapeDtypeStruct((B, D), jnp.float32),
        ),
    )(pooled_view, w_lin1, b_lin1, w_lin2, b_lin2)
    lin1_out, lin2_out, softmax_out, p0, p1, p2, p3 = head_outs
    mlps_output_embeds_pooled = [p0, p1, p2, p3][:L + 1]

    mlps_output_embeds = [e0, e1, e2, e3][:L + 1]

    return (mlps_output_embeds, embeds_stacked, mlps_output_embeds_pooled,
            pooled_view, lin1_out, lin1_out, lin2_out, softmax_out)
